# Initial kernel scaffold; baseline (speedup 1.0000x reference)
import jax
import jax.numpy as jnp
from jax.experimental import pallas as pl


def kernel(user, pos, neg, edge_index, user_emb, item_emb):
    raise NotImplementedError("write your pallas kernel here")


# speedup vs baseline: 14.3387x; 14.3387x over previous
import functools

import jax
import jax.numpy as jnp
from jax import lax
from jax.experimental import pallas as pl
from jax.experimental.pallas import tpu as pltpu
from jax.experimental.pallas import tpu_sc as plsc

N_USERS = 50000
N_ITEMS = 50000
N_NODES = N_USERS + N_ITEMS
HALF = 50000
D = 32
E = 800000
BATCH = 4096
REG = 1e-4

NC = 2
NS = 16
EPT = E // NS
ST = 3128
ST_LAST = HALF - (NS - 1) * ST
CH = 128
NFULL = EPT // CH
TAIL = EPT - NFULL * CH

_mesh = lambda: plsc.VectorSubcoreMesh(core_axis_name="c", subcore_axis_name="s")
_SC_PARAMS = pltpu.CompilerParams(use_tc_tiling_on_sc=False)


def _zero_vmem(ref, nrows):
    z = jnp.zeros((16,), jnp.float32)

    def body(i, _):
        r = i // 2
        col = (i % 2) * 16
        ref[r, pl.ds(col, 16)] = z
        return 0

    lax.fori_loop(0, nrows * 2, body, 0)


def _fill_vmem16(ref, nrows, value):
    v = jnp.full((16,), value, jnp.float32)

    def body(i, _):
        ref[i, pl.ds(0, 16)] = v
        return 0

    lax.fori_loop(0, nrows, body, 0)


def _per_stripe(s, do):
    @pl.when(s < NS - 1)
    def _():
        do(s * ST, ST)

    @pl.when(s == NS - 1)
    def _():
        do((NS - 1) * ST, ST_LAST)


def _zero_rows(acc, zbuf, row0, nrows):
    nfull = nrows // CH

    def body(i, _):
        pltpu.sync_copy(zbuf, acc.at[pl.ds(row0 + i * CH, CH), :])
        return 0

    lax.fori_loop(0, nfull, body, 0)
    rem = nrows - nfull * CH
    if rem:
        pltpu.sync_copy(zbuf.at[pl.ds(0, rem), :],
                        acc.at[pl.ds(row0 + nfull * CH, rem), :])


def _sc_degree_body(dst2, deg_out, ones_v, zbuf, didx, didx_t, acc):
    c = lax.axis_index("c")
    s = lax.axis_index("s")

    _fill_vmem16(ones_v, CH, 1.0)
    _fill_vmem16(zbuf, CH, 0.0)
    _per_stripe(s, lambda row0, nrows: _zero_rows(acc, zbuf, row0, nrows))
    plsc.subcore_barrier()

    def body(t, _):
        base = c * E + s * EPT + t * CH
        pltpu.sync_copy(dst2.at[pl.ds(base, CH)], didx)
        pltpu.sync_copy(ones_v, acc.at[didx], add=True)
        return 0

    lax.fori_loop(0, NFULL, body, 0)
    if TAIL:
        base = c * E + s * EPT + NFULL * CH
        pltpu.sync_copy(dst2.at[pl.ds(base, TAIL)], didx_t)
        pltpu.sync_copy(ones_v.at[pl.ds(0, TAIL), :], acc.at[didx_t], add=True)
    plsc.subcore_barrier()

    def writeout(row0, nrows):
        pltpu.sync_copy(acc.at[pl.ds(row0, nrows), :],
                        deg_out.at[pl.ds(c * HALF + row0, nrows), :])

    _per_stripe(s, writeout)


def _sc_degree(dst2):
    return pl.kernel(
        _sc_degree_body,
        out_type=jax.ShapeDtypeStruct((N_NODES, 16), jnp.float32),
        mesh=_mesh(),
        compiler_params=_SC_PARAMS,
        scratch_types=[
            pltpu.VMEM((CH, 16), jnp.float32),
            pltpu.VMEM((CH, 16), jnp.float32),
            pltpu.VMEM((CH,), jnp.int32),
            pltpu.VMEM((TAIL,), jnp.int32),
            pltpu.VMEM_SHARED((HALF, 16), jnp.float32),
        ],
    )(dst2)


def _sc_layer(g, src2, dst2):
    def body(g_hbm, src2_hbm, dst2_hbm, s_out,
             zbuf, sidx, didx, rows, sidx_t, didx_t, rows_t, acc, sem):
        c = lax.axis_index("c")
        s = lax.axis_index("s")

        _zero_vmem(zbuf, CH)
        _per_stripe(s, lambda row0, nrows: _zero_rows(acc, zbuf, row0, nrows))
        plsc.subcore_barrier()

        def body_t(t, _):
            base = c * E + s * EPT + t * CH
            pltpu.sync_copy(src2_hbm.at[pl.ds(base, CH)], sidx)
            pltpu.sync_copy(dst2_hbm.at[pl.ds(base, CH)], didx)
            pltpu.async_copy(g_hbm.at[sidx], rows, sem).wait()
            pltpu.sync_copy(rows, acc.at[didx], add=True)
            return 0

        lax.fori_loop(0, NFULL, body_t, 0)
        if TAIL:
            base = c * E + s * EPT + NFULL * CH
            pltpu.sync_copy(src2_hbm.at[pl.ds(base, TAIL)], sidx_t)
            pltpu.sync_copy(dst2_hbm.at[pl.ds(base, TAIL)], didx_t)
            pltpu.async_copy(g_hbm.at[sidx_t], rows_t, sem).wait()
            pltpu.sync_copy(rows_t, acc.at[didx_t], add=True)
        plsc.subcore_barrier()

        def writeout(row0, nrows):
            pltpu.sync_copy(acc.at[pl.ds(row0, nrows), :],
                            s_out.at[pl.ds(c * HALF + row0, nrows), :])

        _per_stripe(s, writeout)

    return pl.kernel(
        body,
        out_type=jax.ShapeDtypeStruct((N_NODES, D), jnp.float32),
        mesh=_mesh(),
        compiler_params=_SC_PARAMS,
        scratch_types=[
            pltpu.VMEM((CH, D), jnp.float32),
            pltpu.VMEM((CH,), jnp.int32),
            pltpu.VMEM((CH,), jnp.int32),
            pltpu.VMEM((CH, D), jnp.float32),
            pltpu.VMEM((TAIL,), jnp.int32),
            pltpu.VMEM((TAIL,), jnp.int32),
            pltpu.VMEM((TAIL, D), jnp.float32),
            pltpu.VMEM_SHARED((HALF, D), jnp.float32),
            pltpu.SemaphoreType.DMA,
        ],
    )(g, src2, dst2)


GPW = BATCH // (NC * NS)


def _sc_gather6_body(light, emb0, u_i, p_i, n_i,
                     o_ul, o_pl, o_nl, o_u0, o_p0, o_n0,
                     iu, ip, in_, ipg, ing, rows, sem):
    c = lax.axis_index("c")
    s = lax.axis_index("s")
    w = s * NC + c
    b0 = w * GPW

    pltpu.sync_copy(u_i.at[pl.ds(b0, GPW)], iu)
    pltpu.sync_copy(p_i.at[pl.ds(b0, GPW)], ip)
    pltpu.sync_copy(n_i.at[pl.ds(b0, GPW)], in_)

    def shift(srcr, dstr):
        def body(j, _):
            dstr[pl.ds(j * 16, 16)] = srcr[pl.ds(j * 16, 16)] + N_USERS
            return 0
        lax.fori_loop(0, GPW // 16, body, 0)

    shift(ip, ipg)
    shift(in_, ing)

    def fetch(table, idx, out):
        pltpu.async_copy(table.at[idx], rows, sem).wait()
        pltpu.sync_copy(rows, out.at[pl.ds(b0, GPW), :])

    fetch(light, iu, o_ul)
    fetch(light, ipg, o_pl)
    fetch(light, ing, o_nl)
    fetch(emb0, iu, o_u0)
    fetch(emb0, ipg, o_p0)
    fetch(emb0, ing, o_n0)


def _sc_gather6(light, emb0, user, pos, neg):
    out = jax.ShapeDtypeStruct((BATCH, D), jnp.float32)
    return pl.kernel(
        _sc_gather6_body,
        out_type=(out,) * 6,
        mesh=_mesh(),
        compiler_params=_SC_PARAMS,
        scratch_types=[
            pltpu.VMEM((GPW,), jnp.int32),
            pltpu.VMEM((GPW,), jnp.int32),
            pltpu.VMEM((GPW,), jnp.int32),
            pltpu.VMEM((GPW,), jnp.int32),
            pltpu.VMEM((GPW,), jnp.int32),
            pltpu.VMEM((GPW, D), jnp.float32),
            pltpu.SemaphoreType.DMA,
        ],
    )(light, emb0, user, pos, neg)


RB = 1000
NBLK = N_NODES // RB


def _n_of(deg_blk):
    deg = jnp.sum(deg_blk, axis=1, keepdims=True)
    return lax.rsqrt(jnp.maximum(deg, 1.0))


def _tc_g0_body(emb0_ref, deg_ref, g0_ref):
    g0_ref[...] = emb0_ref[...] * _n_of(deg_ref[...])


def _tc_g0(emb0, deg16):
    return pl.pallas_call(
        _tc_g0_body,
        grid=(NBLK,),
        in_specs=[
            pl.BlockSpec((RB, D), lambda i: (i, 0)),
            pl.BlockSpec((RB, 16), lambda i: (i, 0)),
        ],
        out_specs=pl.BlockSpec((RB, D), lambda i: (i, 0)),
        out_shape=jax.ShapeDtypeStruct((N_NODES, D), jnp.float32),
    )(emb0, deg16)


def _tc_scale_body(s_ref, deg_ref, accp_ref, acc_ref, g_ref):
    n = _n_of(deg_ref[...])
    e = n * s_ref[...]
    acc_ref[...] = accp_ref[...] + e
    g_ref[...] = n * e


def _tc_scale(s_l, deg16, acc_prev):
    shp = jax.ShapeDtypeStruct((N_NODES, D), jnp.float32)
    return pl.pallas_call(
        _tc_scale_body,
        grid=(NBLK,),
        in_specs=[
            pl.BlockSpec((RB, D), lambda i: (i, 0)),
            pl.BlockSpec((RB, 16), lambda i: (i, 0)),
            pl.BlockSpec((RB, D), lambda i: (i, 0)),
        ],
        out_specs=[
            pl.BlockSpec((RB, D), lambda i: (i, 0)),
            pl.BlockSpec((RB, D), lambda i: (i, 0)),
        ],
        out_shape=[shp, shp],
    )(s_l, deg16, acc_prev)


def _tc_final_body(s_ref, deg_ref, accp_ref, light_ref):
    n = _n_of(deg_ref[...])
    light_ref[...] = (accp_ref[...] + n * s_ref[...]) * 0.25


def _tc_final(s_l, deg16, acc_prev):
    return pl.pallas_call(
        _tc_final_body,
        grid=(NBLK,),
        in_specs=[
            pl.BlockSpec((RB, D), lambda i: (i, 0)),
            pl.BlockSpec((RB, 16), lambda i: (i, 0)),
            pl.BlockSpec((RB, D), lambda i: (i, 0)),
        ],
        out_specs=pl.BlockSpec((RB, D), lambda i: (i, 0)),
        out_shape=jax.ShapeDtypeStruct((N_NODES, D), jnp.float32),
    )(s_l, deg16, acc_prev)


def _tc_loss_body(ul_ref, pl_ref, nl_ref, u0_ref, p0_ref, n0_ref, out_ref):
    ul = ul_ref[...]
    ps = jnp.sum(ul * pl_ref[...], axis=1, keepdims=True)
    ns = jnp.sum(ul * nl_ref[...], axis=1, keepdims=True)
    x = ns - ps
    sp = jnp.maximum(x, 0.0) + jnp.log(1.0 + jnp.exp(-jnp.abs(x)))
    reg = REG * (jnp.sum(jnp.abs(u0_ref[...]))
                 + jnp.sum(jnp.abs(p0_ref[...]))
                 + jnp.sum(jnp.abs(n0_ref[...])))
    out_ref[...] = sp + reg


def _tc_loss(ul, plat, nl, u0, p0, n0):
    return pl.pallas_call(
        _tc_loss_body,
        out_shape=jax.ShapeDtypeStruct((BATCH, 1), jnp.float32),
    )(ul, plat, nl, u0, p0, n0)


def kernel(user, pos, neg, edge_index, user_emb, item_emb):
    user = user.astype(jnp.int32)
    pos = pos.astype(jnp.int32)
    neg = neg.astype(jnp.int32)
    e_u = edge_index[0].astype(jnp.int32)
    e_i = edge_index[1].astype(jnp.int32)
    src2 = jnp.concatenate([e_i, e_u])
    dst2 = jnp.concatenate([e_u, e_i - N_USERS])
    emb0 = jnp.concatenate([user_emb, item_emb], axis=0)

    deg16 = _sc_degree(dst2)
    g = _tc_g0(emb0, deg16)
    acc = emb0
    for _ in range(2):
        s_l = _sc_layer(g, src2, dst2)
        acc, g = _tc_scale(s_l, deg16, acc)
    s_l = _sc_layer(g, src2, dst2)
    light = _tc_final(s_l, deg16, acc)

    ul, plat, nl, u0, p0, n0 = _sc_gather6(light, emb0, user, pos, neg)
    return _tc_loss(ul, plat, nl, u0, p0, n0).reshape(BATCH)


# speedup vs baseline: 21.5839x; 1.5053x over previous
import functools

import jax
import jax.numpy as jnp
from jax import lax
from jax.experimental import pallas as pl
from jax.experimental.pallas import tpu as pltpu
from jax.experimental.pallas import tpu_sc as plsc

N_USERS = 50000
N_ITEMS = 50000
N_NODES = N_USERS + N_ITEMS
HALF = 50000
D = 32
E = 800000
BATCH = 4096
REG = 1e-4

NC = 2
NS = 16
CH = 96
GCH = 4
E2 = 811008
EPT = E2 // NS
GROUPS = EPT // (CH * GCH)
EROWS = 2 * E2 // CH
ACC_ROWS = 50048
JUNK = HALF
ST = ACC_ROWS // NS
ST_LAST = HALF - (NS - 1) * ST
ZCH = 128

_mesh = lambda: plsc.VectorSubcoreMesh(core_axis_name="c", subcore_axis_name="s")
_SC_PARAMS = pltpu.CompilerParams(use_tc_tiling_on_sc=False)


def _fill_vmem16(ref, nrows, value):
    v = jnp.full((16,), value, jnp.float32)

    def body(i, _):
        ref[i, pl.ds(0, 16)] = v
        return 0

    lax.fori_loop(0, nrows, body, 0)


def _per_stripe(s, do):
    @pl.when(s < NS - 1)
    def _():
        do(s * ST, ST)

    @pl.when(s == NS - 1)
    def _():
        do((NS - 1) * ST, ST_LAST)


def _zero_rows(acc, zbuf, row0, nrows):
    nfull = nrows // ZCH

    def body(i, _):
        pltpu.sync_copy(zbuf, acc.at[pl.ds(row0 + i * ZCH, ZCH), :])
        return 0

    lax.fori_loop(0, nfull, body, 0)
    rem = nrows - nfull * ZCH
    if rem:
        pltpu.sync_copy(zbuf.at[pl.ds(0, rem), :],
                        acc.at[pl.ds(row0 + nfull * ZCH, rem), :])


def _sc_degree_body(dst2, deg_out, ones_v, zbuf, didx, acc):
    c = lax.axis_index("c")
    s = lax.axis_index("s")

    _fill_vmem16(ones_v, CH, 1.0)
    _fill_vmem16(zbuf, ZCH, 0.0)
    _zero_rows(acc, zbuf, s * ST, ST)
    plsc.subcore_barrier()

    rbase0 = c * (E2 // CH) + s * (EPT // CH)

    def body(g, _):
        pltpu.sync_copy(dst2.at[pl.ds(rbase0 + g * GCH, GCH), :], didx)
        for b in range(GCH):
            pltpu.sync_copy(ones_v, acc.at[didx.at[b]], add=True)
        return 0

    lax.fori_loop(0, GROUPS, body, 0)
    plsc.subcore_barrier()

    def writeout(row0, nrows):
        pltpu.sync_copy(acc.at[pl.ds(row0, nrows), :],
                        deg_out.at[pl.ds(c * HALF + row0, nrows), :])

    _per_stripe(s, writeout)


def _sc_degree(dst2):
    return pl.kernel(
        _sc_degree_body,
        out_type=jax.ShapeDtypeStruct((N_NODES, 16), jnp.float32),
        mesh=_mesh(),
        compiler_params=_SC_PARAMS,
        scratch_types=[
            pltpu.VMEM((CH, 16), jnp.float32),
            pltpu.VMEM((ZCH, 16), jnp.float32),
            pltpu.VMEM((GCH, CH), jnp.int32),
            pltpu.VMEM_SHARED((ACC_ROWS, 16), jnp.float32),
        ],
    )(dst2)


def _sc_layer(g, src2, dst2):
    def body(g_hbm, src2_hbm, dst2_hbm, s_out,
             sidx, didx, rows, acc, sem0, sem1):
        c = lax.axis_index("c")
        s = lax.axis_index("s")
        sems = (sem0, sem1)
        rbase0 = c * (E2 // CH) + s * (EPT // CH)

        def gather_desc(slot, b):
            return pltpu.make_async_copy(
                g_hbm.at[sidx.at[slot, b]],
                rows.at[slot, pl.ds(b * CH, CH), :],
                sems[slot])

        def fetch_group(slot, grp):
            rb = rbase0 + grp * GCH
            pltpu.sync_copy(src2_hbm.at[pl.ds(rb, GCH), :], sidx.at[slot])
            pltpu.sync_copy(dst2_hbm.at[pl.ds(rb, GCH), :], didx.at[slot])
            for b in range(GCH):
                gather_desc(slot, b).start()

        def drain_scatter(slot):
            for b in range(GCH):
                gather_desc(slot, b).wait()
                pltpu.sync_copy(rows.at[slot, pl.ds(b * CH, CH), :],
                                acc.at[didx.at[slot, b]], add=True)

        z = jnp.zeros((16,), jnp.float32)

        def zfill(i, _):
            rows[0, i // 2, pl.ds((i % 2) * 16, 16)] = z
            return 0

        lax.fori_loop(0, ZCH * 2, zfill, 0)
        _zero_rows(acc, rows.at[0, pl.ds(0, ZCH), :], s * ST, ST)
        fetch_group(0, 0)
        fetch_group(1, 1)
        plsc.subcore_barrier()

        def main(i, _):
            for slot in range(2):
                drain_scatter(slot)
                fetch_group(slot, 2 * i + slot + 2)
            return 0

        lax.fori_loop(0, GROUPS // 2 - 1, main, 0)
        drain_scatter(0)
        drain_scatter(1)
        plsc.subcore_barrier()

        def writeout(row0, nrows):
            pltpu.sync_copy(acc.at[pl.ds(row0, nrows), :],
                            s_out.at[pl.ds(c * HALF + row0, nrows), :])

        _per_stripe(s, writeout)

    return pl.kernel(
        body,
        out_type=jax.ShapeDtypeStruct((N_NODES, D), jnp.float32),
        mesh=_mesh(),
        compiler_params=_SC_PARAMS,
        scratch_types=[
            pltpu.VMEM((2, GCH, CH), jnp.int32),
            pltpu.VMEM((2, GCH, CH), jnp.int32),
            pltpu.VMEM((2, GCH * CH, D), jnp.float32),
            pltpu.VMEM_SHARED((ACC_ROWS, D), jnp.float32),
            pltpu.SemaphoreType.DMA,
            pltpu.SemaphoreType.DMA,
        ],
    )(g, src2, dst2)


GPW = BATCH // (NC * NS)


def _sc_gather6_body(light, emb0, u_i, p_i, n_i,
                     o_ul, o_pl, o_nl, o_u0, o_p0, o_n0,
                     iu, ip, in_, ipg, ing, rows, sem):
    c = lax.axis_index("c")
    s = lax.axis_index("s")
    w = s * NC + c
    b0 = w * GPW

    pltpu.sync_copy(u_i.at[pl.ds(b0, GPW)], iu)
    pltpu.sync_copy(p_i.at[pl.ds(b0, GPW)], ip)
    pltpu.sync_copy(n_i.at[pl.ds(b0, GPW)], in_)

    def shift(srcr, dstr):
        def body(j, _):
            dstr[pl.ds(j * 16, 16)] = srcr[pl.ds(j * 16, 16)] + N_USERS
            return 0
        lax.fori_loop(0, GPW // 16, body, 0)

    shift(ip, ipg)
    shift(in_, ing)

    def fetch(table, idx, out):
        pltpu.async_copy(table.at[idx], rows, sem).wait()
        pltpu.sync_copy(rows, out.at[pl.ds(b0, GPW), :])

    fetch(light, iu, o_ul)
    fetch(light, ipg, o_pl)
    fetch(light, ing, o_nl)
    fetch(emb0, iu, o_u0)
    fetch(emb0, ipg, o_p0)
    fetch(emb0, ing, o_n0)


def _sc_gather6(light, emb0, user, pos, neg):
    out = jax.ShapeDtypeStruct((BATCH, D), jnp.float32)
    return pl.kernel(
        _sc_gather6_body,
        out_type=(out,) * 6,
        mesh=_mesh(),
        compiler_params=_SC_PARAMS,
        scratch_types=[
            pltpu.VMEM((GPW,), jnp.int32),
            pltpu.VMEM((GPW,), jnp.int32),
            pltpu.VMEM((GPW,), jnp.int32),
            pltpu.VMEM((GPW,), jnp.int32),
            pltpu.VMEM((GPW,), jnp.int32),
            pltpu.VMEM((GPW, D), jnp.float32),
            pltpu.SemaphoreType.DMA,
        ],
    )(light, emb0, user, pos, neg)


RB = 1000
NBLK = N_NODES // RB


def _n_of(deg_blk):
    deg = jnp.sum(deg_blk, axis=1, keepdims=True)
    return lax.rsqrt(jnp.maximum(deg, 1.0))


def _tc_g0_body(emb0_ref, deg_ref, g0_ref):
    g0_ref[...] = emb0_ref[...] * _n_of(deg_ref[...])


def _tc_g0(emb0, deg16):
    return pl.pallas_call(
        _tc_g0_body,
        grid=(NBLK,),
        in_specs=[
            pl.BlockSpec((RB, D), lambda i: (i, 0)),
            pl.BlockSpec((RB, 16), lambda i: (i, 0)),
        ],
        out_specs=pl.BlockSpec((RB, D), lambda i: (i, 0)),
        out_shape=jax.ShapeDtypeStruct((N_NODES, D), jnp.float32),
    )(emb0, deg16)


def _tc_scale_body(s_ref, deg_ref, accp_ref, acc_ref, g_ref):
    n = _n_of(deg_ref[...])
    e = n * s_ref[...]
    acc_ref[...] = accp_ref[...] + e
    g_ref[...] = n * e


def _tc_scale(s_l, deg16, acc_prev):
    shp = jax.ShapeDtypeStruct((N_NODES, D), jnp.float32)
    return pl.pallas_call(
        _tc_scale_body,
        grid=(NBLK,),
        in_specs=[
            pl.BlockSpec((RB, D), lambda i: (i, 0)),
            pl.BlockSpec((RB, 16), lambda i: (i, 0)),
            pl.BlockSpec((RB, D), lambda i: (i, 0)),
        ],
        out_specs=[
            pl.BlockSpec((RB, D), lambda i: (i, 0)),
            pl.BlockSpec((RB, D), lambda i: (i, 0)),
        ],
        out_shape=[shp, shp],
    )(s_l, deg16, acc_prev)


def _tc_final_body(s_ref, deg_ref, accp_ref, light_ref):
    n = _n_of(deg_ref[...])
    light_ref[...] = (accp_ref[...] + n * s_ref[...]) * 0.25


def _tc_final(s_l, deg16, acc_prev):
    return pl.pallas_call(
        _tc_final_body,
        grid=(NBLK,),
        in_specs=[
            pl.BlockSpec((RB, D), lambda i: (i, 0)),
            pl.BlockSpec((RB, 16), lambda i: (i, 0)),
            pl.BlockSpec((RB, D), lambda i: (i, 0)),
        ],
        out_specs=pl.BlockSpec((RB, D), lambda i: (i, 0)),
        out_shape=jax.ShapeDtypeStruct((N_NODES, D), jnp.float32),
    )(s_l, deg16, acc_prev)


def _tc_loss_body(ul_ref, pl_ref, nl_ref, u0_ref, p0_ref, n0_ref, out_ref):
    ul = ul_ref[...]
    ps = jnp.sum(ul * pl_ref[...], axis=1, keepdims=True)
    ns = jnp.sum(ul * nl_ref[...], axis=1, keepdims=True)
    x = ns - ps
    sp = jnp.maximum(x, 0.0) + jnp.log(1.0 + jnp.exp(-jnp.abs(x)))
    reg = REG * (jnp.sum(jnp.abs(u0_ref[...]))
                 + jnp.sum(jnp.abs(p0_ref[...]))
                 + jnp.sum(jnp.abs(n0_ref[...])))
    out_ref[...] = sp + reg


def _tc_loss(ul, plat, nl, u0, p0, n0):
    return pl.pallas_call(
        _tc_loss_body,
        out_shape=jax.ShapeDtypeStruct((BATCH, 1), jnp.float32),
    )(ul, plat, nl, u0, p0, n0)


def kernel(user, pos, neg, edge_index, user_emb, item_emb):
    user = user.astype(jnp.int32)
    pos = pos.astype(jnp.int32)
    neg = neg.astype(jnp.int32)
    e_u = edge_index[0].astype(jnp.int32)
    e_i = edge_index[1].astype(jnp.int32)
    pad0 = jnp.zeros((E2 - E,), jnp.int32)
    padj = jnp.full((E2 - E,), JUNK, jnp.int32)
    src2 = jnp.concatenate([e_i, pad0, e_u, pad0]).reshape(EROWS, CH)
    dst2 = jnp.concatenate([e_u, padj, e_i - N_USERS, padj]).reshape(EROWS, CH)
    emb0 = jnp.concatenate([user_emb, item_emb], axis=0)

    deg16 = _sc_degree(dst2)
    g = _tc_g0(emb0, deg16)
    acc = emb0
    for _ in range(2):
        s_l = _sc_layer(g, src2, dst2)
        acc, g = _tc_scale(s_l, deg16, acc)
    s_l = _sc_layer(g, src2, dst2)
    light = _tc_final(s_l, deg16, acc)

    ul, plat, nl, u0, p0, n0 = _sc_gather6(light, emb0, user, pos, neg)
    return _tc_loss(ul, plat, nl, u0, p0, n0).reshape(BATCH)


# speedup vs baseline: 21.8570x; 1.0127x over previous
import functools

import jax
import jax.numpy as jnp
from jax import lax
from jax.experimental import pallas as pl
from jax.experimental.pallas import tpu as pltpu
from jax.experimental.pallas import tpu_sc as plsc

N_USERS = 50000
N_ITEMS = 50000
N_NODES = N_USERS + N_ITEMS
HALF = 50000
D = 32
E = 800000
BATCH = 4096
REG = 1e-4

NC = 2
NS = 16
CH = 128
GCH = 3
E2 = 811008
EPT = E2 // NS
GROUPS = EPT // (CH * GCH)
EROWS = 2 * E2 // CH
ACC_ROWS = 50048
JUNK = HALF
ST = ACC_ROWS // NS
ST_LAST = HALF - (NS - 1) * ST
ZCH = 128

_mesh = lambda: plsc.VectorSubcoreMesh(core_axis_name="c", subcore_axis_name="s")
_SC_PARAMS = pltpu.CompilerParams(use_tc_tiling_on_sc=False)


def _fill_vmem16(ref, nrows, value):
    v = jnp.full((16,), value, jnp.float32)

    def body(i, _):
        ref[i, pl.ds(0, 16)] = v
        return 0

    lax.fori_loop(0, nrows, body, 0)


def _per_stripe(s, do):
    @pl.when(s < NS - 1)
    def _():
        do(s * ST, ST)

    @pl.when(s == NS - 1)
    def _():
        do((NS - 1) * ST, ST_LAST)


def _zero_rows(acc, zbuf, row0, nrows):
    nfull = nrows // ZCH

    def body(i, _):
        pltpu.sync_copy(zbuf, acc.at[pl.ds(row0 + i * ZCH, ZCH), :])
        return 0

    lax.fori_loop(0, nfull, body, 0)
    rem = nrows - nfull * ZCH
    if rem:
        pltpu.sync_copy(zbuf.at[pl.ds(0, rem), :],
                        acc.at[pl.ds(row0 + nfull * ZCH, rem), :])


def _sc_degree_body(dst2, deg_out, ones_v, zbuf, didx, acc):
    c = lax.axis_index("c")
    s = lax.axis_index("s")

    _fill_vmem16(ones_v, CH, 1.0)
    _fill_vmem16(zbuf, ZCH, 0.0)
    _zero_rows(acc, zbuf, s * ST, ST)
    plsc.subcore_barrier()

    rbase0 = c * (E2 // CH) + s * (EPT // CH)

    def body(g, _):
        pltpu.sync_copy(dst2.at[pl.ds(rbase0 + g * GCH, GCH), :], didx)
        for b in range(GCH):
            pltpu.sync_copy(ones_v, acc.at[didx.at[b]], add=True)
        return 0

    lax.fori_loop(0, GROUPS, body, 0)
    plsc.subcore_barrier()

    def writeout(row0, nrows):
        pltpu.sync_copy(acc.at[pl.ds(row0, nrows), :],
                        deg_out.at[pl.ds(c * HALF + row0, nrows), :])

    _per_stripe(s, writeout)


def _sc_degree(dst2):
    return pl.kernel(
        _sc_degree_body,
        out_type=jax.ShapeDtypeStruct((N_NODES, 16), jnp.float32),
        mesh=_mesh(),
        compiler_params=_SC_PARAMS,
        scratch_types=[
            pltpu.VMEM((CH, 16), jnp.float32),
            pltpu.VMEM((ZCH, 16), jnp.float32),
            pltpu.VMEM((GCH, CH), jnp.int32),
            pltpu.VMEM_SHARED((ACC_ROWS, 16), jnp.float32),
        ],
    )(dst2)


def _sc_layer(g, src2, dst2):
    def body(g_hbm, src2_hbm, dst2_hbm, s_out,
             sidx, didx, rows, acc, sem0, sem1):
        c = lax.axis_index("c")
        s = lax.axis_index("s")
        sems = (sem0, sem1)
        rbase0 = c * (E2 // CH) + s * (EPT // CH)

        def gather_desc(slot, b):
            return pltpu.make_async_copy(
                g_hbm.at[sidx.at[slot, b]],
                rows.at[slot, pl.ds(b * CH, CH), :],
                sems[slot])

        def fetch_group(slot, grp):
            rb = rbase0 + grp * GCH
            pltpu.sync_copy(src2_hbm.at[pl.ds(rb, GCH), :], sidx.at[slot])
            pltpu.sync_copy(dst2_hbm.at[pl.ds(rb, GCH), :], didx.at[slot])
            for b in range(GCH):
                gather_desc(slot, b).start()

        def drain_scatter(slot):
            for b in range(GCH):
                gather_desc(slot, b).wait()
                pltpu.sync_copy(rows.at[slot, pl.ds(b * CH, CH), :],
                                acc.at[didx.at[slot, b]], add=True)

        z = jnp.zeros((16,), jnp.float32)

        def zfill(i, _):
            rows[0, i // 2, pl.ds((i % 2) * 16, 16)] = z
            return 0

        lax.fori_loop(0, ZCH * 2, zfill, 0)
        _zero_rows(acc, rows.at[0, pl.ds(0, ZCH), :], s * ST, ST)
        fetch_group(0, 0)
        fetch_group(1, 1)
        plsc.subcore_barrier()

        def main(i, _):
            for slot in range(2):
                drain_scatter(slot)
                fetch_group(slot, 2 * i + slot + 2)
            return 0

        lax.fori_loop(0, GROUPS // 2 - 1, main, 0)
        drain_scatter(0)
        drain_scatter(1)
        plsc.subcore_barrier()

        def writeout(row0, nrows):
            pltpu.sync_copy(acc.at[pl.ds(row0, nrows), :],
                            s_out.at[pl.ds(c * HALF + row0, nrows), :])

        _per_stripe(s, writeout)

    return pl.kernel(
        body,
        out_type=jax.ShapeDtypeStruct((N_NODES, D), jnp.float32),
        mesh=_mesh(),
        compiler_params=_SC_PARAMS,
        scratch_types=[
            pltpu.VMEM((2, GCH, CH), jnp.int32),
            pltpu.VMEM((2, GCH, CH), jnp.int32),
            pltpu.VMEM((2, GCH * CH, D), jnp.float32),
            pltpu.VMEM_SHARED((ACC_ROWS, D), jnp.float32),
            pltpu.SemaphoreType.DMA,
            pltpu.SemaphoreType.DMA,
        ],
    )(g, src2, dst2)


GPW = BATCH // (NC * NS)


def _sc_gather6_body(light, emb0, u_i, p_i, n_i,
                     o_ul, o_pl, o_nl, o_u0, o_p0, o_n0,
                     iu, ip, in_, ipg, ing, rows, sem):
    c = lax.axis_index("c")
    s = lax.axis_index("s")
    w = s * NC + c
    b0 = w * GPW

    pltpu.sync_copy(u_i.at[pl.ds(b0, GPW)], iu)
    pltpu.sync_copy(p_i.at[pl.ds(b0, GPW)], ip)
    pltpu.sync_copy(n_i.at[pl.ds(b0, GPW)], in_)

    def shift(srcr, dstr):
        def body(j, _):
            dstr[pl.ds(j * 16, 16)] = srcr[pl.ds(j * 16, 16)] + N_USERS
            return 0
        lax.fori_loop(0, GPW // 16, body, 0)

    shift(ip, ipg)
    shift(in_, ing)

    def fetch(table, idx, out):
        pltpu.async_copy(table.at[idx], rows, sem).wait()
        pltpu.sync_copy(rows, out.at[pl.ds(b0, GPW), :])

    fetch(light, iu, o_ul)
    fetch(light, ipg, o_pl)
    fetch(light, ing, o_nl)
    fetch(emb0, iu, o_u0)
    fetch(emb0, ipg, o_p0)
    fetch(emb0, ing, o_n0)


def _sc_gather6(light, emb0, user, pos, neg):
    out = jax.ShapeDtypeStruct((BATCH, D), jnp.float32)
    return pl.kernel(
        _sc_gather6_body,
        out_type=(out,) * 6,
        mesh=_mesh(),
        compiler_params=_SC_PARAMS,
        scratch_types=[
            pltpu.VMEM((GPW,), jnp.int32),
            pltpu.VMEM((GPW,), jnp.int32),
            pltpu.VMEM((GPW,), jnp.int32),
            pltpu.VMEM((GPW,), jnp.int32),
            pltpu.VMEM((GPW,), jnp.int32),
            pltpu.VMEM((GPW, D), jnp.float32),
            pltpu.SemaphoreType.DMA,
        ],
    )(light, emb0, user, pos, neg)


RB = 1000
NBLK = N_NODES // RB


def _n_of(deg_blk):
    deg = jnp.sum(deg_blk, axis=1, keepdims=True)
    return lax.rsqrt(jnp.maximum(deg, 1.0))


def _tc_g0_body(emb0_ref, deg_ref, g0_ref):
    g0_ref[...] = emb0_ref[...] * _n_of(deg_ref[...])


def _tc_g0(emb0, deg16):
    return pl.pallas_call(
        _tc_g0_body,
        grid=(NBLK,),
        in_specs=[
            pl.BlockSpec((RB, D), lambda i: (i, 0)),
            pl.BlockSpec((RB, 16), lambda i: (i, 0)),
        ],
        out_specs=pl.BlockSpec((RB, D), lambda i: (i, 0)),
        out_shape=jax.ShapeDtypeStruct((N_NODES, D), jnp.float32),
    )(emb0, deg16)


def _tc_scale_body(s_ref, deg_ref, accp_ref, acc_ref, g_ref):
    n = _n_of(deg_ref[...])
    e = n * s_ref[...]
    acc_ref[...] = accp_ref[...] + e
    g_ref[...] = n * e


def _tc_scale(s_l, deg16, acc_prev):
    shp = jax.ShapeDtypeStruct((N_NODES, D), jnp.float32)
    return pl.pallas_call(
        _tc_scale_body,
        grid=(NBLK,),
        in_specs=[
            pl.BlockSpec((RB, D), lambda i: (i, 0)),
            pl.BlockSpec((RB, 16), lambda i: (i, 0)),
            pl.BlockSpec((RB, D), lambda i: (i, 0)),
        ],
        out_specs=[
            pl.BlockSpec((RB, D), lambda i: (i, 0)),
            pl.BlockSpec((RB, D), lambda i: (i, 0)),
        ],
        out_shape=[shp, shp],
    )(s_l, deg16, acc_prev)


def _tc_final_body(s_ref, deg_ref, accp_ref, light_ref):
    n = _n_of(deg_ref[...])
    light_ref[...] = (accp_ref[...] + n * s_ref[...]) * 0.25


def _tc_final(s_l, deg16, acc_prev):
    return pl.pallas_call(
        _tc_final_body,
        grid=(NBLK,),
        in_specs=[
            pl.BlockSpec((RB, D), lambda i: (i, 0)),
            pl.BlockSpec((RB, 16), lambda i: (i, 0)),
            pl.BlockSpec((RB, D), lambda i: (i, 0)),
        ],
        out_specs=pl.BlockSpec((RB, D), lambda i: (i, 0)),
        out_shape=jax.ShapeDtypeStruct((N_NODES, D), jnp.float32),
    )(s_l, deg16, acc_prev)


def _tc_loss_body(ul_ref, pl_ref, nl_ref, u0_ref, p0_ref, n0_ref, out_ref):
    ul = ul_ref[...]
    ps = jnp.sum(ul * pl_ref[...], axis=1, keepdims=True)
    ns = jnp.sum(ul * nl_ref[...], axis=1, keepdims=True)
    x = ns - ps
    sp = jnp.maximum(x, 0.0) + jnp.log(1.0 + jnp.exp(-jnp.abs(x)))
    reg = REG * (jnp.sum(jnp.abs(u0_ref[...]))
                 + jnp.sum(jnp.abs(p0_ref[...]))
                 + jnp.sum(jnp.abs(n0_ref[...])))
    out_ref[...] = sp + reg


def _tc_loss(ul, plat, nl, u0, p0, n0):
    return pl.pallas_call(
        _tc_loss_body,
        out_shape=jax.ShapeDtypeStruct((BATCH, 1), jnp.float32),
    )(ul, plat, nl, u0, p0, n0)


def kernel(user, pos, neg, edge_index, user_emb, item_emb):
    user = user.astype(jnp.int32)
    pos = pos.astype(jnp.int32)
    neg = neg.astype(jnp.int32)
    e_u = edge_index[0].astype(jnp.int32)
    e_i = edge_index[1].astype(jnp.int32)
    pad0 = jnp.zeros((E2 - E,), jnp.int32)
    padj = jnp.full((E2 - E,), JUNK, jnp.int32)
    src2 = jnp.concatenate([e_i, pad0, e_u, pad0]).reshape(EROWS, CH)
    dst2 = jnp.concatenate([e_u, padj, e_i - N_USERS, padj]).reshape(EROWS, CH)
    emb0 = jnp.concatenate([user_emb, item_emb], axis=0)

    deg16 = _sc_degree(dst2)
    g = _tc_g0(emb0, deg16)
    acc = emb0
    for _ in range(2):
        s_l = _sc_layer(g, src2, dst2)
        acc, g = _tc_scale(s_l, deg16, acc)
    s_l = _sc_layer(g, src2, dst2)
    light = _tc_final(s_l, deg16, acc)

    ul, plat, nl, u0, p0, n0 = _sc_gather6(light, emb0, user, pos, neg)
    return _tc_loss(ul, plat, nl, u0, p0, n0).reshape(BATCH)
